# baseline (speedup 1.0000x reference)
import functools

import jax
import jax.numpy as jnp
from jax import lax
from jax.experimental import pallas as pl
from jax.experimental.pallas import tpu as pltpu
from jax.experimental.pallas import tpu_sc as plsc

N = 1000000
_NUM_IDX = 65536
_CHUNK = 128
_NC, _NS = 2, 16
_NW = _NC * _NS
_ROWS_PER_W = _NUM_IDX // (_NW * _CHUNK)

_mesh = plsc.VectorSubcoreMesh(core_axis_name="c", subcore_axis_name="s")


_IDX_PER_S = _NUM_IDX // _NS
_NPAD = 1000448
_HALF = _NPAD // _NC
_REG = _HALF // _NS
_DUMP = 512


@functools.partial(
    pl.kernel,
    out_type=jax.ShapeDtypeStruct((_NPAD,), jnp.float32),
    mesh=_mesh,
    scratch_types=[
        pltpu.VMEM((_IDX_PER_S,), jnp.int32),
        pltpu.VMEM((_IDX_PER_S,), jnp.int32),
        pltpu.VMEM((_IDX_PER_S,), jnp.float32),
        pltpu.VMEM((_REG,), jnp.float32),
        pltpu.VMEM_SHARED((_HALF + _DUMP,), jnp.float32),
        pltpu.SemaphoreType.DMA,
    ],
)
def _sc_build_mask(idx_hbm, mask_out, idx_v, rel_v, neg_v, ones_v, shared,
                   sem):
  c = lax.axis_index("c")
  s = lax.axis_index("s")

  pltpu.sync_copy(idx_hbm.at[pl.ds(s * _IDX_PER_S, _IDX_PER_S)], idx_v)

  def fill(i, carry):
    ones_v[pl.ds(i * 16, 16)] = jnp.ones((16,), jnp.float32)
    return carry
  lax.fori_loop(0, _REG // 16, fill, 0)

  pltpu.sync_copy(ones_v, shared.at[pl.ds(s * _REG, _REG)])

  base = c * _HALF
  def relcalc(i, carry):
    sl = pl.ds(i * 16, 16)
    vec = idx_v[sl]
    rel = vec - base
    inr = (rel >= 0) & (rel < _HALF)
    rel_v[sl] = jnp.where(inr, rel, _HALF + (vec & (_DUMP - 1)))
    neg_v[sl] = jnp.full((16,), -1.0, jnp.float32)
    return carry
  lax.fori_loop(0, _IDX_PER_S // 16, relcalc, 0)

  plsc.subcore_barrier()

  pltpu.sync_copy(neg_v, shared.at[rel_v], add=True)

  plsc.subcore_barrier()

  pltpu.sync_copy(shared.at[pl.ds(s * _REG, _REG)], ones_v)
  pltpu.sync_copy(ones_v, mask_out.at[pl.ds(base + s * _REG, _REG)])


_BL = 98304


def _tc_body(mask_ref, m_ref, s_ref, q_ref, r_ref, o_ref, p_ref,
             mo_ref, so_ref, qo_ref, ro_ref, oo_ref, po_ref):
  raw = mask_ref[...]
  mk = jnp.maximum(raw, 0.0)
  mo_ref[...] = m_ref[...] * mk
  so_ref[...] = s_ref[...] * mk
  qo_ref[...] = q_ref[...] * mk
  ro_ref[...] = r_ref[...] * mk
  oo_ref[...] = o_ref[...] * mk
  po_ref[...] = jnp.where(raw > 0.0, p_ref[...], jnp.zeros_like(p_ref))


def _tc_masked(mask2, mt, st, qt, rt, ot, pt):
  bs = lambda k: pl.BlockSpec((k, _BL), lambda i: (0, i))
  return pl.pallas_call(
      _tc_body,
      grid=(pl.cdiv(N, _BL),),
      in_specs=[bs(1), bs(3), bs(3), bs(4), bs(15), bs(1), bs(1)],
      out_specs=[bs(3), bs(3), bs(4), bs(15), bs(1), bs(1)],
      out_shape=[
          jax.ShapeDtypeStruct((3, N), jnp.float32),
          jax.ShapeDtypeStruct((3, N), jnp.float32),
          jax.ShapeDtypeStruct((4, N), jnp.float32),
          jax.ShapeDtypeStruct((15, N), jnp.float32),
          jax.ShapeDtypeStruct((1, N), jnp.float32),
          jax.ShapeDtypeStruct((1, N), jnp.int8),
      ],
  )(mask2, mt, st, qt, rt, ot, pt)


def kernel(means_3d, scales, quats, rgbs, opacities, persistent_mask, indices):
  mask = _sc_build_mask(indices.astype(jnp.int32))

  pt = persistent_mask.astype(jnp.int8).reshape(1, N)
  outs = _tc_masked(mask.reshape(1, _NPAD), means_3d.T, scales.T, quats.T,
                    rgbs.T, opacities.T, pt)
  return (outs[0].T, outs[1].T, outs[2].T, outs[3].T, outs[4].T,
          outs[5].reshape(N) != 0)


# speedup vs baseline: 1.0131x; 1.0131x over previous
import functools

import jax
import jax.numpy as jnp
from jax import lax
from jax.experimental import pallas as pl
from jax.experimental.pallas import tpu as pltpu
from jax.experimental.pallas import tpu_sc as plsc

N = 1000000
_NUM_IDX = 65536
_NC, _NS = 2, 16
_IDX_PER_S = _NUM_IDX // _NS
_DUMP = 512

_BL = 98304
_NBLK = 11
_SPLIT_BLKS = 5
_SPLIT = _SPLIT_BLKS * _BL
_SIZE_A = _SPLIT
_SIZE_B = 508928

_mesh = plsc.VectorSubcoreMesh(core_axis_name="c", subcore_axis_name="s")


def _make_sc_mask(lo, size):
  qtr = size // _NC
  reg = qtr // _NS
  assert qtr * _NC == size and reg * _NS == qtr and reg % 16 == 0

  @functools.partial(
      pl.kernel,
      out_type=jax.ShapeDtypeStruct((size,), jnp.float32),
      mesh=_mesh,
      scratch_types=[
          pltpu.VMEM((_IDX_PER_S,), jnp.int32),
          pltpu.VMEM((_IDX_PER_S,), jnp.int32),
          pltpu.VMEM((_IDX_PER_S,), jnp.float32),
          pltpu.VMEM((reg,), jnp.float32),
          pltpu.VMEM_SHARED((qtr + _DUMP,), jnp.float32),
          pltpu.SemaphoreType.DMA,
      ],
  )
  def sc_mask(idx_hbm, mask_out, idx_v, rel_v, neg_v, ones_v, shared, sem):
    c = lax.axis_index("c")
    s = lax.axis_index("s")

    pltpu.sync_copy(idx_hbm.at[pl.ds(s * _IDX_PER_S, _IDX_PER_S)], idx_v)

    def fill(i, carry):
      ones_v[pl.ds(i * 16, 16)] = jnp.ones((16,), jnp.float32)
      return carry
    lax.fori_loop(0, reg // 16, fill, 0)

    pltpu.sync_copy(ones_v, shared.at[pl.ds(s * reg, reg)])

    base = lo + c * qtr
    def relcalc(i, carry):
      sl = pl.ds(i * 16, 16)
      vec = idx_v[sl]
      rel = vec - base
      inr = (rel >= 0) & (rel < qtr)
      rel_v[sl] = jnp.where(inr, rel, qtr + (vec & (_DUMP - 1)))
      neg_v[sl] = jnp.full((16,), -1.0, jnp.float32)
      return carry
    lax.fori_loop(0, _IDX_PER_S // 16, relcalc, 0)

    plsc.subcore_barrier()

    pltpu.sync_copy(neg_v, shared.at[rel_v], add=True)

    plsc.subcore_barrier()

    pltpu.sync_copy(shared.at[pl.ds(s * reg, reg)], ones_v)
    pltpu.sync_copy(ones_v, mask_out.at[pl.ds(c * qtr + s * reg, reg)])

  return sc_mask


_sc_mask_a = _make_sc_mask(0, _SIZE_A)
_sc_mask_b = _make_sc_mask(_SPLIT, _SIZE_B)

_OUT_SHAPES = [
    jax.ShapeDtypeStruct((3, N), jnp.float32),
    jax.ShapeDtypeStruct((3, N), jnp.float32),
    jax.ShapeDtypeStruct((4, N), jnp.float32),
    jax.ShapeDtypeStruct((15, N), jnp.float32),
    jax.ShapeDtypeStruct((1, N), jnp.float32),
    jax.ShapeDtypeStruct((1, N), jnp.float32),
]


def _tc_body(mask_ref, m_ref, s_ref, q_ref, r_ref, o_ref, p_ref,
             mo_ref, so_ref, qo_ref, ro_ref, oo_ref, po_ref):
  mk = jnp.maximum(mask_ref[...], 0.0)
  mo_ref[...] = m_ref[...] * mk
  so_ref[...] = s_ref[...] * mk
  qo_ref[...] = q_ref[...] * mk
  ro_ref[...] = r_ref[...] * mk
  oo_ref[...] = o_ref[...] * mk
  po_ref[...] = p_ref[...] * mk


def _tc_body2(mask_ref, m_ref, s_ref, q_ref, r_ref, o_ref, p_ref,
              am_ref, as_ref, aq_ref, ar_ref, ao_ref, ap_ref,
              mo_ref, so_ref, qo_ref, ro_ref, oo_ref, po_ref):
  del am_ref, as_ref, aq_ref, ar_ref, ao_ref, ap_ref
  _tc_body(mask_ref, m_ref, s_ref, q_ref, r_ref, o_ref, p_ref,
           mo_ref, so_ref, qo_ref, ro_ref, oo_ref, po_ref)


def _tc_masked_1(mask2, mt, st, qt, rt, ot, pt):
  bs = lambda k: pl.BlockSpec((k, _BL), lambda i: (0, i))
  return pl.pallas_call(
      _tc_body,
      grid=(_SPLIT_BLKS,),
      in_specs=[bs(1), bs(3), bs(3), bs(4), bs(15), bs(1), bs(1)],
      out_specs=[bs(3), bs(3), bs(4), bs(15), bs(1), bs(1)],
      out_shape=_OUT_SHAPES,
  )(mask2, mt, st, qt, rt, ot, pt)


def _tc_masked_2(mask2, mt, st, qt, rt, ot, pt, prev):
  bs = lambda k: pl.BlockSpec((k, _BL), lambda i: (0, i + _SPLIT_BLKS))
  bm = pl.BlockSpec((1, _BL), lambda i: (0, i))
  ba = pl.BlockSpec(memory_space=pl.ANY)
  return pl.pallas_call(
      _tc_body2,
      grid=(_NBLK - _SPLIT_BLKS,),
      in_specs=[bm, bs(3), bs(3), bs(4), bs(15), bs(1), bs(1),
                ba, ba, ba, ba, ba, ba],
      out_specs=[bs(3), bs(3), bs(4), bs(15), bs(1), bs(1)],
      out_shape=_OUT_SHAPES,
      input_output_aliases={7: 0, 8: 1, 9: 2, 10: 3, 11: 4, 12: 5},
  )(mask2, mt, st, qt, rt, ot, pt, *prev)


def kernel(means_3d, scales, quats, rgbs, opacities, persistent_mask, indices):
  idx = indices.astype(jnp.int32)
  mask_a = _sc_mask_a(idx)
  mask_b = _sc_mask_b(idx)

  pt = persistent_mask.astype(jnp.float32).reshape(1, N)
  views = (means_3d.T, scales.T, quats.T, rgbs.T, opacities.T, pt)
  outs1 = _tc_masked_1(mask_a.reshape(1, _SIZE_A), *views)
  outs = _tc_masked_2(mask_b.reshape(1, _SIZE_B), *views, outs1)
  return (outs[0].T, outs[1].T, outs[2].T, outs[3].T, outs[4].T,
          outs[5].reshape(N) != 0.0)


# speedup vs baseline: 1.0230x; 1.0098x over previous
import functools

import jax
import jax.numpy as jnp
from jax import lax
from jax.experimental import pallas as pl
from jax.experimental.pallas import tpu as pltpu
from jax.experimental.pallas import tpu_sc as plsc

N = 1000000
_NUM_IDX = 65536
_CHUNK = 128
_NC, _NS = 2, 16
_NW = _NC * _NS
_ROWS_PER_W = _NUM_IDX // (_NW * _CHUNK)

_mesh = plsc.VectorSubcoreMesh(core_axis_name="c", subcore_axis_name="s")


_IDX_PER_S = _NUM_IDX // _NS
_NPAD = 1000448
_HALF = _NPAD // _NC
_REG = _HALF // _NS
_DUMP = 512


@functools.partial(
    pl.kernel,
    out_type=jax.ShapeDtypeStruct((_NPAD,), jnp.float32),
    mesh=_mesh,
    scratch_types=[
        pltpu.VMEM((_IDX_PER_S,), jnp.int32),
        pltpu.VMEM((_IDX_PER_S,), jnp.int32),
        pltpu.VMEM((_IDX_PER_S,), jnp.float32),
        pltpu.VMEM((_REG,), jnp.float32),
        pltpu.VMEM_SHARED((_HALF + _DUMP,), jnp.float32),
        pltpu.SemaphoreType.DMA,
    ],
)
def _sc_build_mask(idx_hbm, mask_out, idx_v, rel_v, neg_v, ones_v, shared,
                   sem):
  c = lax.axis_index("c")
  s = lax.axis_index("s")

  pltpu.sync_copy(idx_hbm.at[pl.ds(s * _IDX_PER_S, _IDX_PER_S)], idx_v)

  def fill(i, carry):
    ones_v[pl.ds(i * 16, 16)] = jnp.ones((16,), jnp.float32)
    return carry
  lax.fori_loop(0, _REG // 16, fill, 0)

  pltpu.sync_copy(ones_v, shared.at[pl.ds(s * _REG, _REG)])

  base = c * _HALF
  def relcalc(i, carry):
    sl = pl.ds(i * 16, 16)
    vec = idx_v[sl]
    rel = vec - base
    inr = (rel >= 0) & (rel < _HALF)
    rel_v[sl] = jnp.where(inr, rel, _HALF + (vec & (_DUMP - 1)))
    neg_v[sl] = jnp.full((16,), -1.0, jnp.float32)
    return carry
  lax.fori_loop(0, _IDX_PER_S // 16, relcalc, 0)

  plsc.subcore_barrier()

  pltpu.sync_copy(neg_v, shared.at[rel_v], add=True)

  plsc.subcore_barrier()

  pltpu.sync_copy(shared.at[pl.ds(s * _REG, _REG)], ones_v)
  pltpu.sync_copy(ones_v, mask_out.at[pl.ds(base + s * _REG, _REG)])


_BL = 98304


def _tc_body(mask_ref, m_ref, s_ref, q_ref, r_ref, o_ref, p_ref,
             mo_ref, so_ref, qo_ref, ro_ref, oo_ref, po_ref):
  mk = jnp.maximum(mask_ref[...], 0.0)
  mo_ref[...] = m_ref[...] * mk
  so_ref[...] = s_ref[...] * mk
  qo_ref[...] = q_ref[...] * mk
  ro_ref[...] = r_ref[...] * mk
  oo_ref[...] = o_ref[...] * mk
  po_ref[...] = p_ref[...] * mk


def _tc_masked(mask2, mt, st, qt, rt, ot, pt):
  bs = lambda k: pl.BlockSpec((k, _BL), lambda i: (0, i))
  return pl.pallas_call(
      _tc_body,
      grid=(pl.cdiv(N, _BL),),
      in_specs=[bs(1), bs(3), bs(3), bs(4), bs(15), bs(1), bs(1)],
      out_specs=[bs(3), bs(3), bs(4), bs(15), bs(1), bs(1)],
      out_shape=[
          jax.ShapeDtypeStruct((3, N), jnp.float32),
          jax.ShapeDtypeStruct((3, N), jnp.float32),
          jax.ShapeDtypeStruct((4, N), jnp.float32),
          jax.ShapeDtypeStruct((15, N), jnp.float32),
          jax.ShapeDtypeStruct((1, N), jnp.float32),
          jax.ShapeDtypeStruct((1, N), jnp.float32),
      ],
  )(mask2, mt, st, qt, rt, ot, pt)


def kernel(means_3d, scales, quats, rgbs, opacities, persistent_mask, indices):
  mask = _sc_build_mask(indices.astype(jnp.int32))

  pt = persistent_mask.astype(jnp.float32).reshape(1, N)
  outs = _tc_masked(mask.reshape(1, _NPAD), means_3d.T, scales.T, quats.T,
                    rgbs.T, opacities.T, pt)
  return (outs[0].T, outs[1].T, outs[2].T, outs[3].T, outs[4].T,
          outs[5].reshape(N) != 0.0)
